# baseline (speedup 1.0000x reference)
import functools
import jax
import jax.numpy as jnp
from jax import lax
from jax.experimental import pallas as pl
from jax.experimental.pallas import tpu as pltpu
from jax.experimental.pallas import tpu_sc as plsc

_INFO = plsc.get_sparse_core_info()
_NC, _NS = _INFO.num_cores, _INFO.num_subcores
_NW = _NC * _NS


def kernel(x, pos_table):
    maxlen = x.shape[-1]
    embed_dim = pos_table.shape[-1]
    total = maxlen * embed_dim
    chunk = total // _NW
    flat = pos_table[:maxlen].reshape(total)

    mesh = plsc.VectorSubcoreMesh(core_axis_name="c", subcore_axis_name="s")

    @functools.partial(
        pl.kernel,
        mesh=mesh,
        out_type=jax.ShapeDtypeStruct((total,), pos_table.dtype),
        scratch_types=[pltpu.VMEM((chunk,), pos_table.dtype)],
    )
    def _copy(src_hbm, out_hbm, buf):
        wid = lax.axis_index("s") * _NC + lax.axis_index("c")
        base = wid * chunk
        pltpu.sync_copy(src_hbm.at[pl.ds(base, chunk)], buf)
        pltpu.sync_copy(buf, out_hbm.at[pl.ds(base, chunk)])

    return _copy(flat).reshape(1, maxlen, embed_dim)


# speedup vs baseline: 12.6604x; 12.6604x over previous
import jax
import jax.numpy as jnp
from jax.experimental import pallas as pl


def _embed_kernel(table_ref, out_ref):
    out_ref[0, :, :] = table_ref[:, :]


def kernel(x, pos_table):
    maxlen = x.shape[-1]
    embed_dim = pos_table.shape[-1]
    return pl.pallas_call(
        _embed_kernel,
        out_shape=jax.ShapeDtypeStruct((1, maxlen, embed_dim), pos_table.dtype),
    )(pos_table[:maxlen])
